# baseline (speedup 1.0000x reference)
import numpy as np

import jax
import jax.numpy as jnp
from jax.experimental import pallas as pl
from jax.experimental.pallas import tpu as pltpu

_BN = 8192


def _body(xt_ref, W1s_ref, b1_ref, W2_ref, b2_ref, W3_ref, b3_ref,
          ones_ref, Wc1aT_ref, WscT_ref, bc1e_ref, Wc2T_ref, bc2_ref,
          wc3T_ref, bc3_ref, out_ref):
    xt = xt_ref[...]
    T = xt.shape[0]

    q1 = xt.astype(jnp.bfloat16)
    q2 = q1 * q1
    q3 = q2 * q1
    q4 = q2 * q2

    hs = jnp.dot(W1s_ref[...], q1, preferred_element_type=jnp.float32)
    h = jax.nn.relu(hs[0:512, :].astype(jnp.bfloat16) + b1_ref[...])
    s1 = hs[512:513, :]

    ones_row = ones_ref[...]
    s2 = jnp.dot(ones_row, q2, preferred_element_type=jnp.float32)
    s3 = jnp.dot(ones_row, q3, preferred_element_type=jnp.float32)
    s4 = jnp.dot(ones_row, q4, preferred_element_type=jnp.float32)

    mean = s1 * (1.0 / T)
    var_u = (s2 - mean * s1) * (1.0 / (T - 1))
    std = jnp.sqrt(var_u)
    m3 = (s3 - 3.0 * mean * s2 + 2.0 * mean * mean * s1) * (1.0 / T)
    m4 = (s4 - 4.0 * mean * s3 + 6.0 * mean * mean * s2
          - 3.0 * mean * mean * mean * s1) * (1.0 / T)
    skew = m3 / (std * var_u + 1e-8)
    kurt = m4 / (var_u * var_u + 1e-8)
    mn = jnp.min(q1, axis=0, keepdims=True)
    mx = jnp.max(q1, axis=0, keepdims=True)

    stat6 = jnp.concatenate(
        [mean.astype(jnp.bfloat16), std.astype(jnp.bfloat16), mn, mx,
         skew.astype(jnp.bfloat16), kurt.astype(jnp.bfloat16)], axis=0)

    h = jnp.dot(W2_ref[...], h, preferred_element_type=jnp.float32)
    h = jax.nn.relu(h.astype(jnp.bfloat16) + b2_ref[...])
    h = jnp.dot(W3_ref[...], h, preferred_element_type=jnp.float32)
    seq = jax.nn.relu(h.astype(jnp.bfloat16) + b3_ref[...])

    c = (jnp.dot(Wc1aT_ref[...], seq, preferred_element_type=jnp.float32)
         + jnp.dot(WscT_ref[...], stat6,
                   preferred_element_type=jnp.float32))
    c = jax.nn.relu(c.astype(jnp.bfloat16) + bc1e_ref[...])
    c = jnp.dot(Wc2T_ref[...], c, preferred_element_type=jnp.float32)
    c = jax.nn.relu(c.astype(jnp.bfloat16) + bc2_ref[...])
    z = jnp.dot(wc3T_ref[...], c, preferred_element_type=jnp.float32)
    out_ref[0] = jax.nn.sigmoid(z + bc3_ref[...]) * 4.0 + 6.0


@jax.jit
def kernel(x, W1, b1, W2, b2, W3, b3, Ws, bs, Wc1, bc1, Wc2, bc2, Wc3, bc3):
    B, T = x.shape
    nb = B // _BN
    xt = x.T

    W1sT = jnp.concatenate(
        [W1.T, jnp.ones((1, T), jnp.float32),
         jnp.zeros((7, T), jnp.float32)], axis=0).astype(jnp.bfloat16)
    b1c = b1.reshape(-1, 1).astype(jnp.bfloat16)
    W2T = W2.T.astype(jnp.bfloat16)
    b2c = b2.reshape(-1, 1).astype(jnp.bfloat16)
    W3T = W3.T.astype(jnp.bfloat16)
    b3c = b3.reshape(-1, 1).astype(jnp.bfloat16)
    ones_row = jnp.ones((1, T), jnp.bfloat16)

    Wc1b = Wc1[128:]
    WscT = (Ws @ Wc1b).T.astype(jnp.bfloat16)
    bc1e = (bc1 + bs @ Wc1b).reshape(-1, 1).astype(jnp.bfloat16)
    Wc1aT = Wc1[:128].T.astype(jnp.bfloat16)
    Wc2T = Wc2.T.astype(jnp.bfloat16)
    wc3T = Wc3.T.astype(jnp.bfloat16)
    bc2c = bc2.reshape(-1, 1).astype(jnp.bfloat16)
    bc3c = bc3.reshape(1, 1)

    full = lambda a: pl.BlockSpec(a.shape, lambda i: (0,) * a.ndim)
    out = pl.pallas_call(
        _body,
        grid=(nb,),
        in_specs=[
            pl.BlockSpec((T, _BN), lambda i: (0, i)),
            full(W1sT), full(b1c), full(W2T), full(b2c), full(W3T), full(b3c),
            full(ones_row), full(Wc1aT), full(WscT), full(bc1e),
            full(Wc2T), full(bc2c), full(wc3T), full(bc3c),
        ],
        out_specs=pl.BlockSpec((1, 1, _BN), lambda i: (i, 0, 0)),
        out_shape=jax.ShapeDtypeStruct((nb, 1, _BN), jnp.float32),
        compiler_params=pltpu.CompilerParams(
            dimension_semantics=("parallel",),
        ),
    )(xt, W1sT, b1c, W2T, b2c, W3T, b3c, ones_row,
      Wc1aT, WscT, bc1e, Wc2T, bc2c, wc3T, bc3c)
    return out.reshape(B)


# speedup vs baseline: 1.1551x; 1.1551x over previous
import numpy as np

import jax
import jax.numpy as jnp
from jax.experimental import pallas as pl
from jax.experimental.pallas import tpu as pltpu

_BN = 8192

_R_W1S = 0
_R_W2 = 520
_R_W3 = 776
_R_WC1A = 904
_R_WSC = 968
_R_WC2 = 1032
_R_WC3 = 1064
_WROWS = 1072

_B_B1 = 0
_B_B2 = 512
_B_B3 = 768
_B_BC1 = 896
_B_BC2 = 960
_BROWS = 992


def _body(xt_ref, Wp_ref, bp_ref, ones_ref, bc3_ref, out_ref):
    xt = xt_ref[...]
    T = xt.shape[0]

    q1 = xt.astype(jnp.bfloat16)
    q2 = q1 * q1
    q3 = q2 * q1
    q4 = q2 * q2

    hs = jnp.dot(Wp_ref[_R_W1S:_R_W1S + 520, 0:T], q1,
                 preferred_element_type=jnp.float32)
    h = jax.nn.relu(hs[0:512, :].astype(jnp.bfloat16)
                    + bp_ref[_B_B1:_B_B1 + 512, :])
    s1 = hs[512:513, :]

    ones_row = ones_ref[...]
    s2 = jnp.dot(ones_row, q2, preferred_element_type=jnp.float32)
    s3 = jnp.dot(ones_row, q3, preferred_element_type=jnp.float32)
    s4 = jnp.dot(ones_row, q4, preferred_element_type=jnp.float32)

    mean = s1 * (1.0 / T)
    var_u = (s2 - mean * s1) * (1.0 / (T - 1))
    std = jnp.sqrt(var_u)
    m3 = (s3 - 3.0 * mean * s2 + 2.0 * mean * mean * s1) * (1.0 / T)
    m4 = (s4 - 4.0 * mean * s3 + 6.0 * mean * mean * s2
          - 3.0 * mean * mean * mean * s1) * (1.0 / T)
    skew = m3 / (std * var_u + 1e-8)
    kurt = m4 / (var_u * var_u + 1e-8)
    mn = jnp.min(q1, axis=0, keepdims=True)
    mx = jnp.max(q1, axis=0, keepdims=True)

    stat6 = jnp.concatenate(
        [mean.astype(jnp.bfloat16), std.astype(jnp.bfloat16), mn, mx,
         skew.astype(jnp.bfloat16), kurt.astype(jnp.bfloat16)], axis=0)

    h = jnp.dot(Wp_ref[_R_W2:_R_W2 + 256, 0:512], h,
                preferred_element_type=jnp.float32)
    h = jax.nn.relu(h.astype(jnp.bfloat16)
                    + bp_ref[_B_B2:_B_B2 + 256, :])
    h = jnp.dot(Wp_ref[_R_W3:_R_W3 + 128, 0:256], h,
                preferred_element_type=jnp.float32)
    seq = jax.nn.relu(h.astype(jnp.bfloat16)
                      + bp_ref[_B_B3:_B_B3 + 128, :])

    c = (jnp.dot(Wp_ref[_R_WC1A:_R_WC1A + 64, 0:128], seq,
                 preferred_element_type=jnp.float32)
         + jnp.dot(Wp_ref[_R_WSC:_R_WSC + 64, 0:6], stat6,
                   preferred_element_type=jnp.float32))
    c = jax.nn.relu(c.astype(jnp.bfloat16) + bp_ref[_B_BC1:_B_BC1 + 64, :])
    c = jnp.dot(Wp_ref[_R_WC2:_R_WC2 + 32, 0:64], c,
                preferred_element_type=jnp.float32)
    c = jax.nn.relu(c.astype(jnp.bfloat16) + bp_ref[_B_BC2:_B_BC2 + 32, :])
    z = jnp.dot(Wp_ref[_R_WC3:_R_WC3 + 1, 0:32], c,
                preferred_element_type=jnp.float32)
    out_ref[0] = jax.nn.sigmoid(z + bc3_ref[...]) * 4.0 + 6.0


def _pad(a, rows, cols):
    return jnp.pad(a, ((0, rows - a.shape[0]), (0, cols - a.shape[1])))


@jax.jit
def kernel(x, W1, b1, W2, b2, W3, b3, Ws, bs, Wc1, bc1, Wc2, bc2, Wc3, bc3):
    B, T = x.shape
    nb = B // _BN
    xt = x.T

    Wc1b = Wc1[128:]
    W1s = jnp.concatenate(
        [W1.T, jnp.ones((1, T), jnp.float32)], axis=0)
    Wpack = jnp.concatenate([
        _pad(W1s, 520, 512),
        _pad(W2.T, 256, 512),
        _pad(W3.T, 128, 512),
        _pad(Wc1[:128].T, 64, 512),
        _pad((Ws @ Wc1b).T, 64, 512),
        _pad(Wc2.T, 32, 512),
        _pad(Wc3.T, 8, 512),
    ], axis=0).astype(jnp.bfloat16)
    bpack = jnp.concatenate(
        [b1, b2, b3, bc1 + bs @ Wc1b, bc2]
    ).reshape(-1, 1).astype(jnp.bfloat16)
    ones_row = jnp.asarray(np.ones((1, T)), dtype=jnp.bfloat16)
    bc3c = bc3.reshape(1, 1)

    full = lambda a: pl.BlockSpec(a.shape, lambda i: (0,) * a.ndim)
    out = pl.pallas_call(
        _body,
        grid=(nb,),
        in_specs=[
            pl.BlockSpec((T, _BN), lambda i: (0, i)),
            full(Wpack), full(bpack), full(ones_row), full(bc3c),
        ],
        out_specs=pl.BlockSpec((1, 1, _BN), lambda i: (i, 0, 0)),
        out_shape=jax.ShapeDtypeStruct((nb, 1, _BN), jnp.float32),
        compiler_params=pltpu.CompilerParams(
            dimension_semantics=("parallel",),
        ),
    )(xt, Wpack, bpack, ones_row, bc3c)
    return out.reshape(B)
